# baseline (speedup 1.0000x reference)
import functools
import jax
import jax.numpy as jnp
from jax import lax
from jax.experimental import pallas as pl
from jax.experimental.pallas import tpu as pltpu
from jax.experimental.pallas import tpu_sc as plsc

N = 10000
E = 320000
D = 128
G = 64
NC = 2
NS = 16
NW = NC * NS
EPW = E // NW
CHUNK = 80
NCHUNK = EPW // CHUNK
NP = 10240
RPT = NP // NS
CW = 16


def _sc_mesh():
    return plsc.VectorSubcoreMesh(core_axis_name="c", subcore_axis_name="s")


def _sc_counts(dst, ones, zeros128):
    @functools.partial(
        pl.kernel,
        out_type=[
            jax.ShapeDtypeStruct((NP, D), jnp.float32),
            jax.ShapeDtypeStruct((NP, D), jnp.float32),
        ],
        mesh=_sc_mesh(),
        scratch_types=(
            [pltpu.VMEM((CHUNK,), jnp.int32)] * 4
            + [pltpu.VMEM((CHUNK, D), jnp.float32)]
            + [pltpu.VMEM_SHARED((NP, D), jnp.float32)]
            + [pltpu.SemaphoreType.DMA] * 8
        ),
    )
    def k(dst_hbm, ones_hbm, z_hbm, c0_hbm, c1_hbm, *rest):
        dstb = rest[0:4]
        ones_v = rest[4]
        acc = rest[5]
        isem = rest[6:10]
        ssem = rest[10:14]
        cid = lax.axis_index("c")
        sid = lax.axis_index("s")
        wid = sid * NC + cid
        row = pl.ds(sid * RPT, RPT)
        pltpu.sync_copy(z_hbm.at[row], acc.at[row])
        pltpu.sync_copy(ones_hbm, ones_v)
        plsc.subcore_barrier()

        def step(t, carry):
            dd = []
            for j in range(4):
                off = pl.multiple_of(wid * EPW + (t * 4 + j) * CHUNK, 8)
                d = pltpu.make_async_copy(dst_hbm.at[pl.ds(off, CHUNK)],
                                          dstb[j], isem[j])
                d.start()
                dd.append(d)
            sc = []
            for j in range(4):
                dd[j].wait()
                s = pltpu.make_async_copy(ones_v, acc.at[dstb[j]], ssem[j])
                s.start(add=True)
                sc.append(s)
            for j in range(4):
                sc[j].wait()
            return carry

        lax.fori_loop(0, NCHUNK // 4, step, 0)

        for j in range(NCHUNK - (NCHUNK // 4) * 4):
            off = pl.multiple_of(
                wid * EPW + ((NCHUNK // 4) * 4 + j) * CHUNK, 8)
            pltpu.sync_copy(dst_hbm.at[pl.ds(off, CHUNK)], dstb[0])
            pltpu.sync_copy(ones_v, acc.at[dstb[0]], add=True)

        plsc.subcore_barrier()

        @pl.when(cid == 0)
        def _():
            pltpu.sync_copy(acc.at[row], c0_hbm.at[row])

        @pl.when(cid == 1)
        def _():
            pltpu.sync_copy(acc.at[row], c1_hbm.at[row])

    return k(dst, ones, zeros128)


NBUF = 4
NOUTER = NCHUNK // NBUF
NPEEL = NCHUNK - NOUTER * NBUF


def _sc_agg(h, src, dst, zeros128):
    @functools.partial(
        pl.kernel,
        out_type=[
            jax.ShapeDtypeStruct((NP, D), jnp.float32),
            jax.ShapeDtypeStruct((NP, D), jnp.float32),
        ],
        mesh=_sc_mesh(),
        scratch_types=(
            [pltpu.VMEM((CHUNK,), jnp.int32)] * NBUF
            + [pltpu.VMEM((CHUNK,), jnp.int32)] * NBUF
            + [pltpu.VMEM((CHUNK, D), jnp.float32)] * NBUF
            + [pltpu.VMEM_SHARED((NP, D), jnp.float32)]
            + [pltpu.SemaphoreType.DMA] * (3 * NBUF)
        ),
    )
    def k(h_hbm, src_hbm, dst_hbm, z_hbm, p0_hbm, p1_hbm, *rest):
        srcb = rest[0:NBUF]
        dstb = rest[NBUF:2 * NBUF]
        rows = rest[2 * NBUF:3 * NBUF]
        acc = rest[3 * NBUF]
        isem = rest[3 * NBUF + 1:3 * NBUF + 1 + NBUF]
        gsem = rest[3 * NBUF + 1 + NBUF:3 * NBUF + 1 + 2 * NBUF]
        ssem = rest[3 * NBUF + 1 + 2 * NBUF:3 * NBUF + 1 + 3 * NBUF]
        cid = lax.axis_index("c")
        sid = lax.axis_index("s")
        wid = sid * NC + cid
        row = pl.ds(sid * RPT, RPT)
        pltpu.sync_copy(z_hbm.at[row], acc.at[row])
        plsc.subcore_barrier()

        def step(t, carry):
            @pl.when(t > 0)
            def _():
                for j in range(NBUF):
                    pltpu.make_async_copy(rows[j], acc.at[dstb[j]],
                                          ssem[j]).wait()

            sd = []
            for j in range(NBUF):
                off = pl.multiple_of(wid * EPW + (t * NBUF + j) * CHUNK, 8)
                s = pltpu.make_async_copy(src_hbm.at[pl.ds(off, CHUNK)],
                                          srcb[j], isem[j])
                d = pltpu.make_async_copy(dst_hbm.at[pl.ds(off, CHUNK)],
                                          dstb[j], isem[j])
                s.start()
                d.start()
                sd.append((s, d))
            gd = []
            for j in range(NBUF):
                sd[j][0].wait()
                sd[j][1].wait()
                g = pltpu.make_async_copy(h_hbm.at[srcb[j]], rows[j], gsem[j])
                g.start()
                gd.append(g)
            for j in range(NBUF):
                gd[j].wait()
                pltpu.make_async_copy(rows[j], acc.at[dstb[j]],
                                      ssem[j]).start(add=True)
            return carry

        lax.fori_loop(0, NOUTER, step, 0)
        for j in range(NBUF):
            pltpu.make_async_copy(rows[j], acc.at[dstb[j]], ssem[j]).wait()

        for j in range(NPEEL):
            off = pl.multiple_of(wid * EPW + (NOUTER * NBUF + j) * CHUNK, 8)
            pltpu.sync_copy(src_hbm.at[pl.ds(off, CHUNK)], srcb[0])
            pltpu.sync_copy(dst_hbm.at[pl.ds(off, CHUNK)], dstb[0])
            pltpu.async_copy(h_hbm.at[srcb[0]], rows[0], gsem[0]).wait()
            pltpu.sync_copy(rows[0], acc.at[dstb[0]], add=True)

        plsc.subcore_barrier()

        @pl.when(cid == 0)
        def _():
            pltpu.sync_copy(acc.at[row], p0_hbm.at[row])

        @pl.when(cid == 1)
        def _():
            pltpu.sync_copy(acc.at[row], p1_hbm.at[row])

    return k(h, src, dst, zeros128)


_B = 1000


def _dot(a, b):
    return lax.dot_general(a, b, (((1,), (0,)), ((), ())),
                           preferred_element_type=jnp.float32)


def _tc_conv(p0, p1, c0, c1, h, w_rel, b_rel, w_root, relu):
    def body(p0_ref, p1_ref, c0_ref, c1_ref, h_ref, wr_ref, br_ref, wq_ref,
             o_ref):
        cnt = jnp.maximum(c0_ref[:, 0:1] + c1_ref[:, 0:1], 1.0)
        mean = (p0_ref[...] + p1_ref[...]) / cnt
        out = _dot(mean, wr_ref[...]) + br_ref[...] + _dot(h_ref[...],
                                                           wq_ref[...])
        if relu:
            out = jnp.maximum(out, 0.0)
        o_ref[...] = out

    return pl.pallas_call(
        body,
        grid=(N // _B,),
        in_specs=[
            pl.BlockSpec((_B, D), lambda i: (i, 0)),
            pl.BlockSpec((_B, D), lambda i: (i, 0)),
            pl.BlockSpec((_B, D), lambda i: (i, 0)),
            pl.BlockSpec((_B, D), lambda i: (i, 0)),
            pl.BlockSpec((_B, D), lambda i: (i, 0)),
            pl.BlockSpec((D, D), lambda i: (0, 0)),
            pl.BlockSpec((1, D), lambda i: (0, 0)),
            pl.BlockSpec((D, D), lambda i: (0, 0)),
        ],
        out_specs=pl.BlockSpec((_B, D), lambda i: (i, 0)),
        out_shape=jax.ShapeDtypeStruct((N, D), jnp.float32),
    )(p0, p1, c0, c1, h, w_rel, b_rel, w_root)


def _tc_conv3_pool(p0, p1, c0, c1, h, w_rel, b_rel, w_root, batch2d):
    def body(p0_ref, p1_ref, c0_ref, c1_ref, h_ref, wr_ref, br_ref, wq_ref,
             bat_ref, pool_ref, gcnt_ref):
        i = pl.program_id(0)
        cnt = jnp.maximum(c0_ref[:, 0:1] + c1_ref[:, 0:1], 1.0)
        mean = (p0_ref[...] + p1_ref[...]) / cnt
        hout = _dot(mean, wr_ref[...]) + br_ref[...] + _dot(h_ref[...],
                                                            wq_ref[...])
        onehot = (bat_ref[...] == lax.broadcasted_iota(
            jnp.int32, (_B, G), 1)).astype(jnp.float32)
        psum = lax.dot_general(onehot, hout, (((0,), (0,)), ((), ())),
                               preferred_element_type=jnp.float32)
        csum = lax.dot_general(onehot, jnp.ones((_B, D), jnp.float32),
                               (((0,), (0,)), ((), ())),
                               preferred_element_type=jnp.float32)

        @pl.when(i == 0)
        def _():
            pool_ref[...] = jnp.zeros((G, D), jnp.float32)
            gcnt_ref[...] = jnp.zeros((G, D), jnp.float32)

        pool_ref[...] += psum
        gcnt_ref[...] += csum

    return pl.pallas_call(
        body,
        grid=(N // _B,),
        in_specs=[
            pl.BlockSpec((_B, D), lambda i: (i, 0)),
            pl.BlockSpec((_B, D), lambda i: (i, 0)),
            pl.BlockSpec((_B, D), lambda i: (i, 0)),
            pl.BlockSpec((_B, D), lambda i: (i, 0)),
            pl.BlockSpec((_B, D), lambda i: (i, 0)),
            pl.BlockSpec((D, D), lambda i: (0, 0)),
            pl.BlockSpec((1, D), lambda i: (0, 0)),
            pl.BlockSpec((D, D), lambda i: (0, 0)),
            pl.BlockSpec((_B, 1), lambda i: (i, 0)),
        ],
        out_specs=[
            pl.BlockSpec((G, D), lambda i: (0, 0)),
            pl.BlockSpec((G, D), lambda i: (0, 0)),
        ],
        out_shape=[
            jax.ShapeDtypeStruct((G, D), jnp.float32),
            jax.ShapeDtypeStruct((G, D), jnp.float32),
        ],
    )(p0, p1, c0, c1, h, w_rel, b_rel, w_root, batch2d)


def _tc_head(pool, gcnt, w1, b1, w2, b2, w3, b3):
    def body(pool_ref, gcnt_ref, w1_ref, b1_ref, w2_ref, b2_ref, w3_ref,
             b3_ref, o_ref):
        g = pool_ref[...] / jnp.maximum(gcnt_ref[...], 1.0)
        g = jnp.maximum(_dot(g, w1_ref[...]) + b1_ref[...], 0.0)
        g = jnp.maximum(_dot(g, w2_ref[...]) + b2_ref[...], 0.0)
        o_ref[...] = _dot(g, w3_ref[...]) + b3_ref[...]

    return pl.pallas_call(
        body,
        out_shape=jax.ShapeDtypeStruct((G, G), jnp.float32),
    )(pool, gcnt, w1, b1, w2, b2, w3, b3)


def kernel(x, edge_index, batch, W1_rel, b1_rel, W1_root, W2_rel, b2_rel,
           W2_root, W3_rel, b3_rel, W3_root, W_lin1, b_lin1, W_lin2, b_lin2,
           W_lin, b_lin):
    src = edge_index[0]
    dst = edge_index[1]
    ones = jnp.ones((CHUNK, D), jnp.float32)
    zeros128 = jnp.zeros((NP, D), jnp.float32)
    batch2d = batch.reshape(N, 1)

    c0, c1 = _sc_counts(dst, ones, zeros128)

    p0, p1 = _sc_agg(x, src, dst, zeros128)
    h1 = _tc_conv(p0, p1, c0, c1, x, W1_rel, b1_rel[None, :], W1_root,
                  relu=True)
    p0, p1 = _sc_agg(h1, src, dst, zeros128)
    h2 = _tc_conv(p0, p1, c0, c1, h1, W2_rel, b2_rel[None, :], W2_root,
                  relu=True)
    p0, p1 = _sc_agg(h2, src, dst, zeros128)
    pool, gcnt = _tc_conv3_pool(p0, p1, c0, c1, h2, W3_rel, b3_rel[None, :],
                                W3_root, batch2d)

    return _tc_head(pool, gcnt, W_lin1, b_lin1[None, :], W_lin2,
                    b_lin2[None, :], W_lin, b_lin[None, :])


# speedup vs baseline: 1.0656x; 1.0656x over previous
import functools
import jax
import jax.numpy as jnp
from jax import lax
from jax.experimental import pallas as pl
from jax.experimental.pallas import tpu as pltpu
from jax.experimental.pallas import tpu_sc as plsc

N = 10000
E = 320000
D = 128
G = 64
NC = 2
NS = 16
NW = NC * NS
EPW = E // NW
CHUNK = 80
NCHUNK = EPW // CHUNK
NP = 10240
RPT = NP // NS
CW = 16


def _sc_mesh():
    return plsc.VectorSubcoreMesh(core_axis_name="c", subcore_axis_name="s")


def _sc_counts(dst, ones, zeros128):
    @functools.partial(
        pl.kernel,
        out_type=[
            jax.ShapeDtypeStruct((NP, D), jnp.float32),
            jax.ShapeDtypeStruct((NP, D), jnp.float32),
        ],
        mesh=_sc_mesh(),
        scratch_types=(
            [pltpu.VMEM((CHUNK,), jnp.int32)] * 4
            + [pltpu.VMEM((CHUNK, D), jnp.float32)]
            + [pltpu.VMEM_SHARED((NP, D), jnp.float32)]
            + [pltpu.SemaphoreType.DMA] * 8
        ),
    )
    def k(dst_hbm, ones_hbm, z_hbm, c0_hbm, c1_hbm, *rest):
        dstb = rest[0:4]
        ones_v = rest[4]
        acc = rest[5]
        isem = rest[6:10]
        ssem = rest[10:14]
        cid = lax.axis_index("c")
        sid = lax.axis_index("s")
        wid = sid * NC + cid
        row = pl.ds(sid * RPT, RPT)
        pltpu.sync_copy(z_hbm.at[row], acc.at[row])
        pltpu.sync_copy(ones_hbm, ones_v)
        plsc.subcore_barrier()

        def step(t, carry):
            dd = []
            for j in range(4):
                off = pl.multiple_of(wid * EPW + (t * 4 + j) * CHUNK, 8)
                d = pltpu.make_async_copy(dst_hbm.at[pl.ds(off, CHUNK)],
                                          dstb[j], isem[j])
                d.start()
                dd.append(d)
            sc = []
            for j in range(4):
                dd[j].wait()
                s = pltpu.make_async_copy(ones_v, acc.at[dstb[j]], ssem[j])
                s.start(add=True)
                sc.append(s)
            for j in range(4):
                sc[j].wait()
            return carry

        lax.fori_loop(0, NCHUNK // 4, step, 0)

        for j in range(NCHUNK - (NCHUNK // 4) * 4):
            off = pl.multiple_of(
                wid * EPW + ((NCHUNK // 4) * 4 + j) * CHUNK, 8)
            pltpu.sync_copy(dst_hbm.at[pl.ds(off, CHUNK)], dstb[0])
            pltpu.sync_copy(ones_v, acc.at[dstb[0]], add=True)

        plsc.subcore_barrier()

        @pl.when(cid == 0)
        def _():
            pltpu.sync_copy(acc.at[row], c0_hbm.at[row])

        @pl.when(cid == 1)
        def _():
            pltpu.sync_copy(acc.at[row], c1_hbm.at[row])

    return k(dst, ones, zeros128)


NBUF = 4
NOUTER = NCHUNK // NBUF
NPEEL = NCHUNK - NOUTER * NBUF


def _sc_agg(h, src, dst, zeros128):
    @functools.partial(
        pl.kernel,
        out_type=[
            jax.ShapeDtypeStruct((NP, D), jnp.float32),
            jax.ShapeDtypeStruct((NP, D), jnp.float32),
        ],
        mesh=_sc_mesh(),
        scratch_types=(
            [pltpu.VMEM((CHUNK,), jnp.int32)] * (2 * NBUF)
            + [pltpu.VMEM((CHUNK,), jnp.int32)] * (2 * NBUF)
            + [pltpu.VMEM((CHUNK, D), jnp.float32)] * NBUF
            + [pltpu.VMEM_SHARED((NP, D), jnp.float32)]
            + [pltpu.SemaphoreType.DMA] * (4 * NBUF)
        ),
    )
    def k(h_hbm, src_hbm, dst_hbm, z_hbm, p0_hbm, p1_hbm, *rest):
        srcb = rest[0:2 * NBUF]
        dstb = rest[2 * NBUF:4 * NBUF]
        rows = rest[4 * NBUF:5 * NBUF]
        acc = rest[5 * NBUF]
        isem = rest[5 * NBUF + 1:5 * NBUF + 1 + 2 * NBUF]
        gsem = rest[5 * NBUF + 1 + 2 * NBUF:5 * NBUF + 1 + 3 * NBUF]
        ssem = rest[5 * NBUF + 1 + 3 * NBUF:5 * NBUF + 1 + 4 * NBUF]
        cid = lax.axis_index("c")
        sid = lax.axis_index("s")
        wid = sid * NC + cid
        row = pl.ds(sid * RPT, RPT)
        pltpu.sync_copy(z_hbm.at[row], acc.at[row])
        plsc.subcore_barrier()

        def idx_start(t, grp):
            out = []
            for j in range(NBUF):
                off = pl.multiple_of(wid * EPW + (t * NBUF + j) * CHUNK, 8)
                b = grp * NBUF + j
                s = pltpu.make_async_copy(src_hbm.at[pl.ds(off, CHUNK)],
                                          srcb[b], isem[b])
                d = pltpu.make_async_copy(dst_hbm.at[pl.ds(off, CHUNK)],
                                          dstb[b], isem[b])
                s.start()
                d.start()
                out.append((s, d))
            return out

        def step(t, grp, first, last):
            if not first:
                for j in range(NBUF):
                    pltpu.make_async_copy(rows[j], acc.at[dstb[j]],
                                          ssem[j]).wait()
            gd = []
            for j in range(NBUF):
                b = grp * NBUF + j
                pltpu.make_async_copy(src_hbm.at[pl.ds(0, CHUNK)], srcb[b],
                                      isem[b]).wait()
                pltpu.make_async_copy(dst_hbm.at[pl.ds(0, CHUNK)], dstb[b],
                                      isem[b]).wait()
                g = pltpu.make_async_copy(h_hbm.at[srcb[b]], rows[j], gsem[j])
                g.start()
                gd.append(g)
            if not last:
                idx_start(t + 1, 1 - grp)
            for j in range(NBUF):
                b = grp * NBUF + j
                gd[j].wait()
                pltpu.make_async_copy(rows[j], acc.at[dstb[b]],
                                      ssem[j]).start(add=True)

        def pair(u, carry):
            step(2 * u + 1, 1, False, False)
            step(2 * u + 2, 0, False, False)
            return carry

        idx_start(0, 0)
        step(0, 0, True, False)
        lax.fori_loop(0, (NOUTER - 3) // 2, pair, 0)
        step(NOUTER - 2, 1, False, False)
        step(NOUTER - 1, 0, False, True)
        for j in range(NBUF):
            pltpu.make_async_copy(rows[j], acc.at[dstb[j]], ssem[j]).wait()

        for j in range(NPEEL):
            off = pl.multiple_of(wid * EPW + (NOUTER * NBUF + j) * CHUNK, 8)
            pltpu.sync_copy(src_hbm.at[pl.ds(off, CHUNK)], srcb[0])
            pltpu.sync_copy(dst_hbm.at[pl.ds(off, CHUNK)], dstb[0])
            pltpu.async_copy(h_hbm.at[srcb[0]], rows[0], gsem[0]).wait()
            pltpu.sync_copy(rows[0], acc.at[dstb[0]], add=True)

        plsc.subcore_barrier()

        @pl.when(cid == 0)
        def _():
            pltpu.sync_copy(acc.at[row], p0_hbm.at[row])

        @pl.when(cid == 1)
        def _():
            pltpu.sync_copy(acc.at[row], p1_hbm.at[row])

    return k(h, src, dst, zeros128)


_B = 1000


def _dot(a, b):
    return lax.dot_general(a, b, (((1,), (0,)), ((), ())),
                           preferred_element_type=jnp.float32)


def _tc_conv(p0, p1, c0, c1, h, w_rel, b_rel, w_root, relu):
    def body(p0_ref, p1_ref, c0_ref, c1_ref, h_ref, wr_ref, br_ref, wq_ref,
             o_ref):
        cnt = jnp.maximum(c0_ref[:, 0:1] + c1_ref[:, 0:1], 1.0)
        mean = (p0_ref[...] + p1_ref[...]) / cnt
        out = _dot(mean, wr_ref[...]) + br_ref[...] + _dot(h_ref[...],
                                                           wq_ref[...])
        if relu:
            out = jnp.maximum(out, 0.0)
        o_ref[...] = out

    return pl.pallas_call(
        body,
        grid=(N // _B,),
        in_specs=[
            pl.BlockSpec((_B, D), lambda i: (i, 0)),
            pl.BlockSpec((_B, D), lambda i: (i, 0)),
            pl.BlockSpec((_B, D), lambda i: (i, 0)),
            pl.BlockSpec((_B, D), lambda i: (i, 0)),
            pl.BlockSpec((_B, D), lambda i: (i, 0)),
            pl.BlockSpec((D, D), lambda i: (0, 0)),
            pl.BlockSpec((1, D), lambda i: (0, 0)),
            pl.BlockSpec((D, D), lambda i: (0, 0)),
        ],
        out_specs=pl.BlockSpec((_B, D), lambda i: (i, 0)),
        out_shape=jax.ShapeDtypeStruct((N, D), jnp.float32),
    )(p0, p1, c0, c1, h, w_rel, b_rel, w_root)


def _tc_conv3_pool(p0, p1, c0, c1, h, w_rel, b_rel, w_root, batch2d):
    def body(p0_ref, p1_ref, c0_ref, c1_ref, h_ref, wr_ref, br_ref, wq_ref,
             bat_ref, pool_ref, gcnt_ref):
        i = pl.program_id(0)
        cnt = jnp.maximum(c0_ref[:, 0:1] + c1_ref[:, 0:1], 1.0)
        mean = (p0_ref[...] + p1_ref[...]) / cnt
        hout = _dot(mean, wr_ref[...]) + br_ref[...] + _dot(h_ref[...],
                                                            wq_ref[...])
        onehot = (bat_ref[...] == lax.broadcasted_iota(
            jnp.int32, (_B, G), 1)).astype(jnp.float32)
        psum = lax.dot_general(onehot, hout, (((0,), (0,)), ((), ())),
                               preferred_element_type=jnp.float32)
        csum = lax.dot_general(onehot, jnp.ones((_B, D), jnp.float32),
                               (((0,), (0,)), ((), ())),
                               preferred_element_type=jnp.float32)

        @pl.when(i == 0)
        def _():
            pool_ref[...] = jnp.zeros((G, D), jnp.float32)
            gcnt_ref[...] = jnp.zeros((G, D), jnp.float32)

        pool_ref[...] += psum
        gcnt_ref[...] += csum

    return pl.pallas_call(
        body,
        grid=(N // _B,),
        in_specs=[
            pl.BlockSpec((_B, D), lambda i: (i, 0)),
            pl.BlockSpec((_B, D), lambda i: (i, 0)),
            pl.BlockSpec((_B, D), lambda i: (i, 0)),
            pl.BlockSpec((_B, D), lambda i: (i, 0)),
            pl.BlockSpec((_B, D), lambda i: (i, 0)),
            pl.BlockSpec((D, D), lambda i: (0, 0)),
            pl.BlockSpec((1, D), lambda i: (0, 0)),
            pl.BlockSpec((D, D), lambda i: (0, 0)),
            pl.BlockSpec((_B, 1), lambda i: (i, 0)),
        ],
        out_specs=[
            pl.BlockSpec((G, D), lambda i: (0, 0)),
            pl.BlockSpec((G, D), lambda i: (0, 0)),
        ],
        out_shape=[
            jax.ShapeDtypeStruct((G, D), jnp.float32),
            jax.ShapeDtypeStruct((G, D), jnp.float32),
        ],
    )(p0, p1, c0, c1, h, w_rel, b_rel, w_root, batch2d)


def _tc_head(pool, gcnt, w1, b1, w2, b2, w3, b3):
    def body(pool_ref, gcnt_ref, w1_ref, b1_ref, w2_ref, b2_ref, w3_ref,
             b3_ref, o_ref):
        g = pool_ref[...] / jnp.maximum(gcnt_ref[...], 1.0)
        g = jnp.maximum(_dot(g, w1_ref[...]) + b1_ref[...], 0.0)
        g = jnp.maximum(_dot(g, w2_ref[...]) + b2_ref[...], 0.0)
        o_ref[...] = _dot(g, w3_ref[...]) + b3_ref[...]

    return pl.pallas_call(
        body,
        out_shape=jax.ShapeDtypeStruct((G, G), jnp.float32),
    )(pool, gcnt, w1, b1, w2, b2, w3, b3)


def kernel(x, edge_index, batch, W1_rel, b1_rel, W1_root, W2_rel, b2_rel,
           W2_root, W3_rel, b3_rel, W3_root, W_lin1, b_lin1, W_lin2, b_lin2,
           W_lin, b_lin):
    src = edge_index[0]
    dst = edge_index[1]
    ones = jnp.ones((CHUNK, D), jnp.float32)
    zeros128 = jnp.zeros((NP, D), jnp.float32)
    batch2d = batch.reshape(N, 1)

    c0, c1 = _sc_counts(dst, ones, zeros128)

    p0, p1 = _sc_agg(x, src, dst, zeros128)
    h1 = _tc_conv(p0, p1, c0, c1, x, W1_rel, b1_rel[None, :], W1_root,
                  relu=True)
    p0, p1 = _sc_agg(h1, src, dst, zeros128)
    h2 = _tc_conv(p0, p1, c0, c1, h1, W2_rel, b2_rel[None, :], W2_root,
                  relu=True)
    p0, p1 = _sc_agg(h2, src, dst, zeros128)
    pool, gcnt = _tc_conv3_pool(p0, p1, c0, c1, h2, W3_rel, b3_rel[None, :],
                                W3_root, batch2d)

    return _tc_head(pool, gcnt, W_lin1, b_lin1[None, :], W_lin2,
                    b_lin2[None, :], W_lin, b_lin[None, :])
